# baseline (device time: 191470 ns/iter reference)
import jax
import jax.numpy as jnp
from jax import lax
from jax.experimental import pallas as pl
from jax.experimental.pallas import tpu as pltpu

N_DEV = 16
FULL_HOPS = 7
N_SEMS = 32


def kernel(x, w_mat):
    m_per, k = x.shape
    _, n_per = w_mat.shape

    def body(x_ref, w_ref, out_ref, comm_ref, send_sems, recv_sems):
        my = lax.axis_index("i")
        left = lax.rem(my + N_DEV - 1, N_DEV)
        right = lax.rem(my + 1, N_DEV)
        half = k // 2

        barrier_sem = pltpu.get_barrier_semaphore()
        for nbr in (left, right):
            pl.semaphore_signal(
                barrier_sem, inc=1,
                device_id=(nbr,), device_id_type=pl.DeviceIdType.MESH,
            )
        pl.semaphore_wait(barrier_sem, 2)

        sends = []

        def start(src_ref, dst_slot, sem_i, dev, col_lo):
            rdma = pltpu.make_async_remote_copy(
                src_ref=src_ref,
                dst_ref=comm_ref.at[dst_slot, :, pl.ds(col_lo, half)],
                send_sem=send_sems.at[sem_i],
                recv_sem=recv_sems.at[sem_i],
                device_id=(dev,),
                device_id_type=pl.DeviceIdType.MESH,
            )
            rdma.start()
            sends.append(rdma)
            return rdma

        def piece(slot, p):
            ref = x_ref if slot is None else comm_ref.at[slot]
            return ref.at[:, pl.ds(p * half, half)]

        def start_r(h, p):
            src = None if h == 0 else h - 1
            return start(piece(src, p), h, 2 * h + p, right, p * half)

        def start_l(h, p):
            src = None if h == 0 else 8 + h - 1
            return start(piece(src, p), 8 + h, 16 + 2 * h + p, left, p * half)

        r_desc = [[None, None] for _ in range(FULL_HOPS)]
        l_desc = [[None, None] for _ in range(FULL_HOPS)]
        for p in (0, 1):
            r_desc[0][p] = start_r(0, p)
            l_desc[0][p] = start_l(0, p)

        out_ref[pl.ds(my * m_per, m_per), :] = jnp.dot(
            x_ref[...], w_ref[...], preferred_element_type=jnp.float32,
        )

        r_last = l_last = None
        for h in range(FULL_HOPS):
            r_desc[h][0].wait_recv()
            if h + 1 < FULL_HOPS:
                r_desc[h + 1][0] = start_r(h + 1, 0)
            else:
                r_last = start(piece(h, 0), 7, 14, right, 0)
            l_desc[h][0].wait_recv()
            if h + 1 < FULL_HOPS:
                l_desc[h + 1][0] = start_l(h + 1, 0)
            r_desc[h][1].wait_recv()
            if h + 1 < FULL_HOPS:
                r_desc[h + 1][1] = start_r(h + 1, 1)
            l_desc[h][1].wait_recv()
            if h + 1 < FULL_HOPS:
                l_desc[h + 1][1] = start_l(h + 1, 1)
            else:
                l_last = start(piece(8 + h, 1), 7, 30, left, half)

            origin_r = lax.rem(my - h - 1 + N_DEV, N_DEV)
            out_ref[pl.ds(origin_r * m_per, m_per), :] = jnp.dot(
                comm_ref[h, :, :], w_ref[...],
                preferred_element_type=jnp.float32,
            )
            origin_l = lax.rem(my + h + 1, N_DEV)
            out_ref[pl.ds(origin_l * m_per, m_per), :] = jnp.dot(
                comm_ref[8 + h, :, :], w_ref[...],
                preferred_element_type=jnp.float32,
            )

        r_last.wait_recv()
        l_last.wait_recv()
        origin_8 = lax.rem(my + N_DEV // 2, N_DEV)
        out_ref[pl.ds(origin_8 * m_per, m_per), :] = jnp.dot(
            comm_ref[7, :, :], w_ref[...],
            preferred_element_type=jnp.float32,
        )

        for rdma in sends:
            rdma.wait_send()

    x16 = x.astype(jnp.bfloat16)
    w16 = w_mat.astype(jnp.bfloat16)
    return pl.pallas_call(
        body,
        out_shape=jax.ShapeDtypeStruct((N_DEV * m_per, n_per), jnp.float32),
        in_specs=[
            pl.BlockSpec(memory_space=pltpu.VMEM),
            pl.BlockSpec(memory_space=pltpu.VMEM),
        ],
        out_specs=pl.BlockSpec(memory_space=pltpu.VMEM),
        scratch_shapes=[
            pltpu.VMEM((N_DEV - 1, m_per, k), jnp.bfloat16),
            pltpu.SemaphoreType.DMA((N_SEMS,)),
            pltpu.SemaphoreType.DMA((N_SEMS,)),
        ],
        compiler_params=pltpu.CompilerParams(
            collective_id=0,
            vmem_limit_bytes=56 * 1024 * 1024,
        ),
    )(x16, w16)


# device time: 186049 ns/iter; 1.0291x vs baseline; 1.0291x over previous
import jax
import jax.numpy as jnp
from jax import lax
from jax.experimental import pallas as pl
from jax.experimental.pallas import tpu as pltpu

N_DEV = 16
FULL_HOPS = 7
N_SEMS = 32


def kernel(x, w_mat):
    m_per, k = x.shape
    _, n_per = w_mat.shape

    def body(x_ref, w_ref, out_ref, comm_ref, w16_ref, send_sems, recv_sems):
        my = lax.axis_index("i")
        left = lax.rem(my + N_DEV - 1, N_DEV)
        right = lax.rem(my + 1, N_DEV)
        half = k // 2

        barrier_sem = pltpu.get_barrier_semaphore()
        for nbr in (left, right):
            pl.semaphore_signal(
                barrier_sem, inc=1,
                device_id=(nbr,), device_id_type=pl.DeviceIdType.MESH,
            )
        pl.semaphore_wait(barrier_sem, 2)

        sends = []

        def start(src_slot, dst_slot, sem_i, dev, col_lo):
            rdma = pltpu.make_async_remote_copy(
                src_ref=comm_ref.at[src_slot, :, pl.ds(col_lo, half)],
                dst_ref=comm_ref.at[dst_slot, :, pl.ds(col_lo, half)],
                send_sem=send_sems.at[sem_i],
                recv_sem=recv_sems.at[sem_i],
                device_id=(dev,),
                device_id_type=pl.DeviceIdType.MESH,
            )
            rdma.start()
            sends.append(rdma)
            return rdma

        def start_r(h, p):
            src = N_DEV - 1 if h == 0 else h - 1
            return start(src, h, 2 * h + p, right, p * half)

        def start_l(h, p):
            src = N_DEV - 1 if h == 0 else 8 + h - 1
            return start(src, 8 + h, 16 + 2 * h + p, left, p * half)

        r_desc = [[None, None] for _ in range(FULL_HOPS)]
        l_desc = [[None, None] for _ in range(FULL_HOPS)]
        for p in (0, 1):
            comm_ref[N_DEV - 1, :, pl.ds(p * half, half)] = x_ref[
                :, pl.ds(p * half, half)
            ].astype(jnp.bfloat16)
            r_desc[0][p] = start_r(0, p)
            l_desc[0][p] = start_l(0, p)

        w16_ref[...] = w_ref[...].astype(jnp.bfloat16)
        out_ref[pl.ds(my * m_per, m_per), :] = jnp.dot(
            comm_ref[N_DEV - 1, :, :], w16_ref[...],
            preferred_element_type=jnp.float32,
        )

        r_last = l_last = None
        for h in range(FULL_HOPS):
            r_desc[h][0].wait_recv()
            if h + 1 < FULL_HOPS:
                r_desc[h + 1][0] = start_r(h + 1, 0)
            else:
                r_last = start(h, 7, 14, right, 0)
            l_desc[h][0].wait_recv()
            if h + 1 < FULL_HOPS:
                l_desc[h + 1][0] = start_l(h + 1, 0)
            r_desc[h][1].wait_recv()
            if h + 1 < FULL_HOPS:
                r_desc[h + 1][1] = start_r(h + 1, 1)
            l_desc[h][1].wait_recv()
            if h + 1 < FULL_HOPS:
                l_desc[h + 1][1] = start_l(h + 1, 1)
            else:
                l_last = start(8 + h, 7, 30, left, half)

            origin_r = lax.rem(my - h - 1 + N_DEV, N_DEV)
            out_ref[pl.ds(origin_r * m_per, m_per), :] = jnp.dot(
                comm_ref[h, :, :], w16_ref[...],
                preferred_element_type=jnp.float32,
            )
            origin_l = lax.rem(my + h + 1, N_DEV)
            out_ref[pl.ds(origin_l * m_per, m_per), :] = jnp.dot(
                comm_ref[8 + h, :, :], w16_ref[...],
                preferred_element_type=jnp.float32,
            )

        r_last.wait_recv()
        l_last.wait_recv()
        origin_8 = lax.rem(my + N_DEV // 2, N_DEV)
        out_ref[pl.ds(origin_8 * m_per, m_per), :] = jnp.dot(
            comm_ref[7, :, :], w16_ref[...],
            preferred_element_type=jnp.float32,
        )

        for rdma in sends:
            rdma.wait_send()

    return pl.pallas_call(
        body,
        out_shape=jax.ShapeDtypeStruct((N_DEV * m_per, n_per), jnp.float32),
        in_specs=[
            pl.BlockSpec(memory_space=pltpu.VMEM),
            pl.BlockSpec(memory_space=pltpu.VMEM),
        ],
        out_specs=pl.BlockSpec(memory_space=pltpu.VMEM),
        scratch_shapes=[
            pltpu.VMEM((N_DEV, m_per, k), jnp.bfloat16),
            pltpu.VMEM((k, n_per), jnp.bfloat16),
            pltpu.SemaphoreType.DMA((N_SEMS,)),
            pltpu.SemaphoreType.DMA((N_SEMS,)),
        ],
        compiler_params=pltpu.CompilerParams(
            collective_id=0,
            vmem_limit_bytes=56 * 1024 * 1024,
        ),
    )(x, w_mat)
